# Initial kernel scaffold; baseline (speedup 1.0000x reference)
import functools

import jax
import jax.numpy as jnp
from jax import lax
from jax.experimental import pallas as pl
from jax.experimental.pallas import tpu as pltpu
from jax.experimental.pallas import tpu_sc as plsc

N = 10000
NPAD = 10240
IN_DIM = 128
HID = 64
OUT = 16

NSC = 2
NTILE = 16
NW = NSC * NTILE
CHUNK = 128
EPT_CHUNKS = 80
EPT = CHUNK * EPT_CHUNKS
EPAD = EPT * NW
ROWS_PT = NPAD // NTILE

BLK = 1000
GRID = N // BLK


def _vsc_mesh():
    return plsc.VectorSubcoreMesh(core_axis_name="c", subcore_axis_name="s")


def _sc_degree(idx):
    @functools.partial(
        pl.kernel,
        out_type=jax.ShapeDtypeStruct((NSC, NPAD, 16), jnp.float32),
        mesh=_vsc_mesh(),
        scratch_types=[
            pltpu.VMEM((1, CHUNK), jnp.int32),
            pltpu.VMEM((CHUNK, 16), jnp.float32),
            pltpu.VMEM_SHARED((NPAD, 16), jnp.float32),
        ],
    )
    def k(idx_hbm, out_hbm, didx, ones, acc):
        c = lax.axis_index("c")
        s = lax.axis_index("s")
        wid = c * NTILE + s

        @pl.loop(0, CHUNK)
        def _(i):
            ones[i, :] = jnp.zeros((16,), jnp.float32)

        @pl.loop(0, ROWS_PT // CHUNK)
        def _(r):
            pltpu.sync_copy(ones.at[:], acc.at[pl.ds(s * ROWS_PT + r * CHUNK, CHUNK)])

        @pl.loop(0, CHUNK)
        def _(i):
            ones[i, :] = jnp.ones((16,), jnp.float32)

        plsc.subcore_barrier()

        @pl.loop(0, EPT_CHUNKS)
        def _(ci):
            pltpu.sync_copy(idx_hbm.at[wid * EPT_CHUNKS + ci, 1], didx.at[0])
            pltpu.sync_copy(ones, acc.at[didx.at[0]], add=True)

        plsc.subcore_barrier()
        pltpu.sync_copy(acc.at[pl.ds(s * ROWS_PT, ROWS_PT)],
                        out_hbm.at[c, pl.ds(s * ROWS_PT, ROWS_PT)])

    return k(idx)


def _sc_aggregate(g, idx):
    @functools.partial(
        pl.kernel,
        out_type=jax.ShapeDtypeStruct((NSC, NPAD, HID), jnp.float32),
        mesh=_vsc_mesh(),
        scratch_types=[
            pltpu.VMEM((2, CHUNK), jnp.int32),
            pltpu.VMEM((CHUNK, HID), jnp.float32),
            pltpu.VMEM((16, HID), jnp.float32),
            pltpu.VMEM_SHARED((NPAD, HID), jnp.float32),
        ],
    )
    def k(g_hbm, idx_hbm, out_hbm, eidx, rows, zbuf, acc):
        c = lax.axis_index("c")
        s = lax.axis_index("s")
        wid = c * NTILE + s

        @pl.loop(0, 16)
        def _(i):
            @pl.loop(0, HID // 16)
            def _(j):
                zbuf[i, pl.ds(j * 16, 16)] = jnp.zeros((16,), jnp.float32)

        @pl.loop(0, ROWS_PT // 16)
        def _(r):
            pltpu.sync_copy(zbuf, acc.at[pl.ds(s * ROWS_PT + r * 16, 16)])

        plsc.subcore_barrier()

        @pl.loop(0, EPT_CHUNKS)
        def _(ci):
            pltpu.sync_copy(idx_hbm.at[wid * EPT_CHUNKS + ci], eidx)
            pltpu.sync_copy(g_hbm.at[eidx.at[0]], rows)
            pltpu.sync_copy(rows, acc.at[eidx.at[1]], add=True)

        plsc.subcore_barrier()
        pltpu.sync_copy(acc.at[pl.ds(s * ROWS_PT, ROWS_PT)],
                        out_hbm.at[c, pl.ds(s * ROWS_PT, ROWS_PT)])

    return k(g, idx)


def _tc_matmul(x, w):
    K, M = w.shape

    def body(x_ref, w_ref, o_ref):
        o_ref[...] = jnp.dot(x_ref[...], w_ref[...],
                             preferred_element_type=jnp.float32)

    return pl.pallas_call(
        body,
        grid=(GRID,),
        in_specs=[pl.BlockSpec((BLK, K), lambda i: (i, 0)),
                  pl.BlockSpec((K, M), lambda i: (0, 0))],
        out_specs=pl.BlockSpec((BLK, M), lambda i: (i, 0)),
        out_shape=jax.ShapeDtypeStruct((N, M), jnp.float32),
    )(x, w)


def _dis_of(degp0, degp1):
    deg = degp0[:, 0:1] + degp1[:, 0:1] + 1.0
    return lax.rsqrt(deg)


def _tc_scale(h, degp):
    M = h.shape[1]

    def body(h_ref, d_ref, o_ref):
        dis = _dis_of(d_ref[0], d_ref[1])
        o_ref[...] = h_ref[...] * dis

    return pl.pallas_call(
        body,
        grid=(GRID,),
        in_specs=[pl.BlockSpec((BLK, M), lambda i: (i, 0)),
                  pl.BlockSpec((NSC, BLK, 16), lambda i: (0, i, 0))],
        out_specs=pl.BlockSpec((BLK, M), lambda i: (i, 0)),
        out_shape=jax.ShapeDtypeStruct((N, M), jnp.float32),
    )(h, degp)


def _tc_combine_matmul_scale(q, g, degp, b, w):
    M = g.shape[1]
    M2 = w.shape[1]

    def body(q_ref, g_ref, d_ref, b_ref, w_ref, o_ref):
        dis = _dis_of(d_ref[0], d_ref[1])
        t = dis * (q_ref[0] + q_ref[1] + g_ref[...]) + b_ref[...]
        t = jnp.maximum(t, 0.0)
        o_ref[...] = jnp.dot(t, w_ref[...],
                             preferred_element_type=jnp.float32) * dis

    return pl.pallas_call(
        body,
        grid=(GRID,),
        in_specs=[pl.BlockSpec((NSC, BLK, M), lambda i: (0, i, 0)),
                  pl.BlockSpec((BLK, M), lambda i: (i, 0)),
                  pl.BlockSpec((NSC, BLK, 16), lambda i: (0, i, 0)),
                  pl.BlockSpec((1, M), lambda i: (0, 0)),
                  pl.BlockSpec((M, M2), lambda i: (0, 0))],
        out_specs=pl.BlockSpec((BLK, M2), lambda i: (i, 0)),
        out_shape=jax.ShapeDtypeStruct((N, M2), jnp.float32),
    )(q, g, degp, b, w)


def _tc_final(q, g, degp, b, wf, bf):
    M = g.shape[1]

    def body(q_ref, g_ref, d_ref, b_ref, w_ref, bf_ref, o_ref):
        dis = _dis_of(d_ref[0], d_ref[1])
        t = dis * (q_ref[0] + q_ref[1] + g_ref[...]) + b_ref[...]
        t = jnp.maximum(t, 0.0)
        logits = jnp.dot(t, w_ref[...],
                         preferred_element_type=jnp.float32) + bf_ref[...]
        m = jnp.max(logits, axis=1, keepdims=True)
        lse = jnp.log(jnp.sum(jnp.exp(logits - m), axis=1, keepdims=True)) + m
        o_ref[...] = logits - lse

    return pl.pallas_call(
        body,
        grid=(GRID,),
        in_specs=[pl.BlockSpec((NSC, BLK, M), lambda i: (0, i, 0)),
                  pl.BlockSpec((BLK, M), lambda i: (i, 0)),
                  pl.BlockSpec((NSC, BLK, 16), lambda i: (0, i, 0)),
                  pl.BlockSpec((1, M), lambda i: (0, 0)),
                  pl.BlockSpec((M, OUT), lambda i: (0, 0)),
                  pl.BlockSpec((1, OUT), lambda i: (0, 0))],
        out_specs=pl.BlockSpec((BLK, OUT), lambda i: (i, 0)),
        out_shape=jax.ShapeDtypeStruct((N, OUT), jnp.float32),
    )(q, g, degp, b, wf, bf)


def kernel(x, edge_index, W1, b1, W2, b2, Wf, bf):
    E = edge_index.shape[1]
    pad = EPAD - E
    src = jnp.concatenate([edge_index[0],
                           jnp.zeros((pad,), jnp.int32)])
    dst = jnp.concatenate([edge_index[1],
                           N + (jnp.arange(pad, dtype=jnp.int32) % (NPAD - N))])
    idx = jnp.stack([src.reshape(-1, CHUNK), dst.reshape(-1, CHUNK)], axis=1)

    degp = _sc_degree(idx)
    degp = degp[:, :N, :]

    h1 = _tc_matmul(x, W1)
    g1 = _tc_scale(h1, degp)

    q1 = _sc_aggregate(g1, idx)[:, :N, :]
    g2 = _tc_combine_matmul_scale(q1, g1, degp, b1.reshape(1, HID), W2)

    q2 = _sc_aggregate(g2, idx)[:, :N, :]
    return _tc_final(q2, g2, degp, b2.reshape(1, HID), Wf, bf.reshape(1, OUT))


# speedup vs baseline: 11.8408x; 11.8408x over previous
import functools

import jax
import jax.numpy as jnp
from jax import lax
from jax.experimental import pallas as pl
from jax.experimental.pallas import tpu as pltpu
from jax.experimental.pallas import tpu_sc as plsc

N = 10000
NPAD = 10240
IN_DIM = 128
HID = 64
OUT = 16

NSC = 2
NTILE = 16
NW = NSC * NTILE
CHUNK = 128
EPT_CHUNKS = 80
EPT = CHUNK * EPT_CHUNKS
EPAD = EPT * NW
ROWS_PT = NPAD // NTILE

BLK = 1000
GRID = N // BLK


def _vsc_mesh():
    return plsc.VectorSubcoreMesh(core_axis_name="c", subcore_axis_name="s")


_SC_PARAMS = pltpu.CompilerParams(use_tc_tiling_on_sc=False)


def _sc_degree(idx):
    @functools.partial(
        pl.kernel,
        out_type=jax.ShapeDtypeStruct((NSC, NPAD, 16), jnp.float32),
        mesh=_vsc_mesh(),
        scratch_types=[
            pltpu.VMEM((1, CHUNK), jnp.int32),
            pltpu.VMEM((CHUNK, 16), jnp.float32),
            pltpu.VMEM_SHARED((NPAD, 16), jnp.float32),
        ],
        compiler_params=_SC_PARAMS,
    )
    def k(idx_hbm, out_hbm, didx, ones, acc):
        c = lax.axis_index("c")
        s = lax.axis_index("s")
        wid = c * NTILE + s

        @pl.loop(0, CHUNK)
        def _(i):
            ones[i, :] = jnp.zeros((16,), jnp.float32)

        @pl.loop(0, ROWS_PT // CHUNK)
        def _(r):
            pltpu.sync_copy(ones.at[:], acc.at[pl.ds(s * ROWS_PT + r * CHUNK, CHUNK)])

        @pl.loop(0, CHUNK)
        def _(i):
            ones[i, :] = jnp.ones((16,), jnp.float32)

        plsc.subcore_barrier()

        @pl.loop(0, EPT_CHUNKS)
        def _(ci):
            pltpu.sync_copy(idx_hbm.at[wid * EPT_CHUNKS + ci, 1], didx.at[0])
            pltpu.sync_copy(ones, acc.at[didx.at[0]], add=True)

        plsc.subcore_barrier()
        pltpu.sync_copy(acc.at[pl.ds(s * ROWS_PT, ROWS_PT)],
                        out_hbm.at[c, pl.ds(s * ROWS_PT, ROWS_PT)])

    return k(idx)


def _sc_aggregate(g, idx):
    @functools.partial(
        pl.kernel,
        out_type=jax.ShapeDtypeStruct((NSC, NPAD, HID), jnp.float32),
        mesh=_vsc_mesh(),
        scratch_types=[
            pltpu.VMEM((2, CHUNK), jnp.int32),
            pltpu.VMEM((CHUNK, HID), jnp.float32),
            pltpu.VMEM((16, HID), jnp.float32),
            pltpu.VMEM_SHARED((NPAD, HID), jnp.float32),
        ],
        compiler_params=_SC_PARAMS,
    )
    def k(g_hbm, idx_hbm, out_hbm, eidx, rows, zbuf, acc):
        c = lax.axis_index("c")
        s = lax.axis_index("s")
        wid = c * NTILE + s

        @pl.loop(0, 16)
        def _(i):
            @pl.loop(0, HID // 16)
            def _(j):
                zbuf[i, pl.ds(j * 16, 16)] = jnp.zeros((16,), jnp.float32)

        @pl.loop(0, ROWS_PT // 16)
        def _(r):
            pltpu.sync_copy(zbuf, acc.at[pl.ds(s * ROWS_PT + r * 16, 16)])

        plsc.subcore_barrier()

        @pl.loop(0, EPT_CHUNKS)
        def _(ci):
            pltpu.sync_copy(idx_hbm.at[wid * EPT_CHUNKS + ci], eidx)
            pltpu.sync_copy(g_hbm.at[eidx.at[0]], rows)
            pltpu.sync_copy(rows, acc.at[eidx.at[1]], add=True)

        plsc.subcore_barrier()
        pltpu.sync_copy(acc.at[pl.ds(s * ROWS_PT, ROWS_PT)],
                        out_hbm.at[c, pl.ds(s * ROWS_PT, ROWS_PT)])

    return k(g, idx)


def _tc_matmul(x, w):
    K, M = w.shape

    def body(x_ref, w_ref, o_ref):
        o_ref[...] = jnp.dot(x_ref[...], w_ref[...],
                             preferred_element_type=jnp.float32)

    return pl.pallas_call(
        body,
        grid=(GRID,),
        in_specs=[pl.BlockSpec((BLK, K), lambda i: (i, 0)),
                  pl.BlockSpec((K, M), lambda i: (0, 0))],
        out_specs=pl.BlockSpec((BLK, M), lambda i: (i, 0)),
        out_shape=jax.ShapeDtypeStruct((N, M), jnp.float32),
    )(x, w)


def _dis_of(degp0, degp1):
    deg = degp0[:, 0:1] + degp1[:, 0:1] + 1.0
    return lax.rsqrt(deg)


def _tc_scale(h, degp):
    M = h.shape[1]

    def body(h_ref, d_ref, o_ref):
        dis = _dis_of(d_ref[0], d_ref[1])
        o_ref[...] = h_ref[...] * dis

    return pl.pallas_call(
        body,
        grid=(GRID,),
        in_specs=[pl.BlockSpec((BLK, M), lambda i: (i, 0)),
                  pl.BlockSpec((NSC, BLK, 16), lambda i: (0, i, 0))],
        out_specs=pl.BlockSpec((BLK, M), lambda i: (i, 0)),
        out_shape=jax.ShapeDtypeStruct((N, M), jnp.float32),
    )(h, degp)


def _tc_combine_matmul_scale(q, g, degp, b, w):
    M = g.shape[1]
    M2 = w.shape[1]

    def body(q_ref, g_ref, d_ref, b_ref, w_ref, o_ref):
        dis = _dis_of(d_ref[0], d_ref[1])
        t = dis * (q_ref[0] + q_ref[1] + g_ref[...]) + b_ref[...]
        t = jnp.maximum(t, 0.0)
        o_ref[...] = jnp.dot(t, w_ref[...],
                             preferred_element_type=jnp.float32) * dis

    return pl.pallas_call(
        body,
        grid=(GRID,),
        in_specs=[pl.BlockSpec((NSC, BLK, M), lambda i: (0, i, 0)),
                  pl.BlockSpec((BLK, M), lambda i: (i, 0)),
                  pl.BlockSpec((NSC, BLK, 16), lambda i: (0, i, 0)),
                  pl.BlockSpec((1, M), lambda i: (0, 0)),
                  pl.BlockSpec((M, M2), lambda i: (0, 0))],
        out_specs=pl.BlockSpec((BLK, M2), lambda i: (i, 0)),
        out_shape=jax.ShapeDtypeStruct((N, M2), jnp.float32),
    )(q, g, degp, b, w)


def _tc_final(q, g, degp, b, wf, bf):
    M = g.shape[1]

    def body(q_ref, g_ref, d_ref, b_ref, w_ref, bf_ref, o_ref):
        dis = _dis_of(d_ref[0], d_ref[1])
        t = dis * (q_ref[0] + q_ref[1] + g_ref[...]) + b_ref[...]
        t = jnp.maximum(t, 0.0)
        logits = jnp.dot(t, w_ref[...],
                         preferred_element_type=jnp.float32) + bf_ref[...]
        m = jnp.max(logits, axis=1, keepdims=True)
        lse = jnp.log(jnp.sum(jnp.exp(logits - m), axis=1, keepdims=True)) + m
        o_ref[...] = logits - lse

    return pl.pallas_call(
        body,
        grid=(GRID,),
        in_specs=[pl.BlockSpec((NSC, BLK, M), lambda i: (0, i, 0)),
                  pl.BlockSpec((BLK, M), lambda i: (i, 0)),
                  pl.BlockSpec((NSC, BLK, 16), lambda i: (0, i, 0)),
                  pl.BlockSpec((1, M), lambda i: (0, 0)),
                  pl.BlockSpec((M, OUT), lambda i: (0, 0)),
                  pl.BlockSpec((1, OUT), lambda i: (0, 0))],
        out_specs=pl.BlockSpec((BLK, OUT), lambda i: (i, 0)),
        out_shape=jax.ShapeDtypeStruct((N, OUT), jnp.float32),
    )(q, g, degp, b, wf, bf)


def kernel(x, edge_index, W1, b1, W2, b2, Wf, bf):
    E = edge_index.shape[1]
    pad = EPAD - E
    src = jnp.concatenate([edge_index[0],
                           jnp.zeros((pad,), jnp.int32)])
    dst = jnp.concatenate([edge_index[1],
                           N + (jnp.arange(pad, dtype=jnp.int32) % (NPAD - N))])
    idx = jnp.stack([src.reshape(-1, CHUNK), dst.reshape(-1, CHUNK)], axis=1)

    degp = _sc_degree(idx)
    degp = degp[:, :N, :]

    h1 = _tc_matmul(x, W1)
    g1 = _tc_scale(h1, degp)

    q1 = _sc_aggregate(g1, idx)[:, :N, :]
    g2 = _tc_combine_matmul_scale(q1, g1, degp, b1.reshape(1, HID), W2)

    q2 = _sc_aggregate(g2, idx)[:, :N, :]
    return _tc_final(q2, g2, degp, b2.reshape(1, HID), Wf, bf.reshape(1, OUT))


# speedup vs baseline: 14.7452x; 1.2453x over previous
import functools

import jax
import jax.numpy as jnp
from jax import lax
from jax.experimental import pallas as pl
from jax.experimental.pallas import tpu as pltpu
from jax.experimental.pallas import tpu_sc as plsc

N = 10000
NPAD = 10240
IN_DIM = 128
HID = 64
OUT = 16

NSC = 2
NTILE = 16
NW = NSC * NTILE
CHUNK = 128
EPT_CHUNKS = 80
EPT = CHUNK * EPT_CHUNKS
EPAD = EPT * NW
ROWS_PT = NPAD // NTILE
NBUF = 8
DEPTH = 4

BLK = 1000
GRID = N // BLK


def _vsc_mesh():
    return plsc.VectorSubcoreMesh(core_axis_name="c", subcore_axis_name="s")


_SC_PARAMS = pltpu.CompilerParams(use_tc_tiling_on_sc=False)


def _sc_degree(idx):
    @functools.partial(
        pl.kernel,
        out_type=jax.ShapeDtypeStruct((NSC, NPAD, 16), jnp.float32),
        mesh=_vsc_mesh(),
        scratch_types=[
            pltpu.VMEM((1, CHUNK), jnp.int32),
            pltpu.VMEM((CHUNK, 16), jnp.float32),
            pltpu.VMEM_SHARED((NPAD, 16), jnp.float32),
        ],
        compiler_params=_SC_PARAMS,
    )
    def k(idx_hbm, out_hbm, didx, ones, acc):
        c = lax.axis_index("c")
        s = lax.axis_index("s")
        wid = c * NTILE + s

        @pl.loop(0, CHUNK)
        def _(i):
            ones[i, :] = jnp.zeros((16,), jnp.float32)

        @pl.loop(0, ROWS_PT // CHUNK)
        def _(r):
            pltpu.sync_copy(ones.at[:], acc.at[pl.ds(s * ROWS_PT + r * CHUNK, CHUNK)])

        @pl.loop(0, CHUNK)
        def _(i):
            ones[i, :] = jnp.ones((16,), jnp.float32)

        plsc.subcore_barrier()

        @pl.loop(0, EPT_CHUNKS)
        def _(ci):
            pltpu.sync_copy(idx_hbm.at[wid * EPT_CHUNKS + ci, 1], didx.at[0])
            pltpu.sync_copy(ones, acc.at[didx.at[0]], add=True)

        plsc.subcore_barrier()
        pltpu.sync_copy(acc.at[pl.ds(s * ROWS_PT, ROWS_PT)],
                        out_hbm.at[c, pl.ds(s * ROWS_PT, ROWS_PT)])

    return k(idx)


def _sc_aggregate(g, idx):
    @functools.partial(
        pl.kernel,
        out_type=jax.ShapeDtypeStruct((NSC, NPAD, HID), jnp.float32),
        mesh=_vsc_mesh(),
        scratch_types=[
            pltpu.VMEM((EPT_CHUNKS, 2, CHUNK), jnp.int32),
            pltpu.VMEM((NBUF, CHUNK, HID), jnp.float32),
            pltpu.VMEM((16, HID), jnp.float32),
            pltpu.VMEM_SHARED((NPAD, HID), jnp.float32),
            pltpu.SemaphoreType.DMA((NBUF,)),
            pltpu.SemaphoreType.DMA((NBUF,)),
        ],
        compiler_params=_SC_PARAMS,
    )
    def k(g_hbm, idx_hbm, out_hbm, idx_all, rows, zbuf, acc, sem_g, sem_s):
        c = lax.axis_index("c")
        s = lax.axis_index("s")
        wid = c * NTILE + s

        pltpu.sync_copy(idx_hbm.at[pl.ds(wid * EPT_CHUNKS, EPT_CHUNKS)], idx_all)

        def start_gather(slot, ci):
            pltpu.async_copy(g_hbm.at[idx_all.at[ci, 0]], rows.at[slot],
                             sem_g.at[slot])

        def wait_gather(slot, ci):
            pltpu.make_async_copy(g_hbm.at[idx_all.at[ci, 0]], rows.at[slot],
                                  sem_g.at[slot]).wait()

        def start_scatter(slot, ci):
            pltpu.async_copy(rows.at[slot], acc.at[idx_all.at[ci, 1]],
                             sem_s.at[slot], add=True)

        def wait_scatter(slot, ci):
            pltpu.make_async_copy(rows.at[slot], acc.at[idx_all.at[ci, 1]],
                                  sem_s.at[slot]).wait()

        for b in range(DEPTH):
            start_gather(b, b)

        @pl.loop(0, 16)
        def _(i):
            @pl.loop(0, HID // 16)
            def _(j):
                zbuf[i, pl.ds(j * 16, 16)] = jnp.zeros((16,), jnp.float32)

        @pl.loop(0, ROWS_PT // 16)
        def _(r):
            pltpu.sync_copy(zbuf, acc.at[pl.ds(s * ROWS_PT + r * 16, 16)])

        plsc.subcore_barrier()

        @pl.loop(0, EPT_CHUNKS, step=NBUF)
        def _(t):
            for b in range(NBUF):
                ci = t + b
                slot_s = b
                wait_gather(slot_s, ci)
                start_scatter(slot_s, ci)
                cg = ci + DEPTH
                slot_g = (b + DEPTH) % NBUF

                @pl.when(cg < EPT_CHUNKS)
                def _():
                    @pl.when(cg >= NBUF)
                    def _():
                        wait_scatter(slot_g, ci)
                    start_gather(slot_g, cg)

        for b in range(NBUF):
            wait_scatter(b, EPT_CHUNKS - NBUF + b)

        plsc.subcore_barrier()
        pltpu.sync_copy(acc.at[pl.ds(s * ROWS_PT, ROWS_PT)],
                        out_hbm.at[c, pl.ds(s * ROWS_PT, ROWS_PT)])

    return k(g, idx)


def _tc_matmul(x, w):
    K, M = w.shape

    def body(x_ref, w_ref, o_ref):
        o_ref[...] = jnp.dot(x_ref[...], w_ref[...],
                             preferred_element_type=jnp.float32)

    return pl.pallas_call(
        body,
        grid=(GRID,),
        in_specs=[pl.BlockSpec((BLK, K), lambda i: (i, 0)),
                  pl.BlockSpec((K, M), lambda i: (0, 0))],
        out_specs=pl.BlockSpec((BLK, M), lambda i: (i, 0)),
        out_shape=jax.ShapeDtypeStruct((N, M), jnp.float32),
    )(x, w)


def _dis_of(degp0, degp1):
    deg = degp0[:, 0:1] + degp1[:, 0:1] + 1.0
    return lax.rsqrt(deg)


def _tc_scale(h, degp):
    M = h.shape[1]

    def body(h_ref, d_ref, o_ref):
        dis = _dis_of(d_ref[0], d_ref[1])
        o_ref[...] = h_ref[...] * dis

    return pl.pallas_call(
        body,
        grid=(GRID,),
        in_specs=[pl.BlockSpec((BLK, M), lambda i: (i, 0)),
                  pl.BlockSpec((NSC, BLK, 16), lambda i: (0, i, 0))],
        out_specs=pl.BlockSpec((BLK, M), lambda i: (i, 0)),
        out_shape=jax.ShapeDtypeStruct((N, M), jnp.float32),
    )(h, degp)


def _tc_combine_matmul_scale(q, g, degp, b, w):
    M = g.shape[1]
    M2 = w.shape[1]

    def body(q_ref, g_ref, d_ref, b_ref, w_ref, o_ref):
        dis = _dis_of(d_ref[0], d_ref[1])
        t = dis * (q_ref[0] + q_ref[1] + g_ref[...]) + b_ref[...]
        t = jnp.maximum(t, 0.0)
        o_ref[...] = jnp.dot(t, w_ref[...],
                             preferred_element_type=jnp.float32) * dis

    return pl.pallas_call(
        body,
        grid=(GRID,),
        in_specs=[pl.BlockSpec((NSC, BLK, M), lambda i: (0, i, 0)),
                  pl.BlockSpec((BLK, M), lambda i: (i, 0)),
                  pl.BlockSpec((NSC, BLK, 16), lambda i: (0, i, 0)),
                  pl.BlockSpec((1, M), lambda i: (0, 0)),
                  pl.BlockSpec((M, M2), lambda i: (0, 0))],
        out_specs=pl.BlockSpec((BLK, M2), lambda i: (i, 0)),
        out_shape=jax.ShapeDtypeStruct((N, M2), jnp.float32),
    )(q, g, degp, b, w)


def _tc_final(q, g, degp, b, wf, bf):
    M = g.shape[1]

    def body(q_ref, g_ref, d_ref, b_ref, w_ref, bf_ref, o_ref):
        dis = _dis_of(d_ref[0], d_ref[1])
        t = dis * (q_ref[0] + q_ref[1] + g_ref[...]) + b_ref[...]
        t = jnp.maximum(t, 0.0)
        logits = jnp.dot(t, w_ref[...],
                         preferred_element_type=jnp.float32) + bf_ref[...]
        m = jnp.max(logits, axis=1, keepdims=True)
        lse = jnp.log(jnp.sum(jnp.exp(logits - m), axis=1, keepdims=True)) + m
        o_ref[...] = logits - lse

    return pl.pallas_call(
        body,
        grid=(GRID,),
        in_specs=[pl.BlockSpec((NSC, BLK, M), lambda i: (0, i, 0)),
                  pl.BlockSpec((BLK, M), lambda i: (i, 0)),
                  pl.BlockSpec((NSC, BLK, 16), lambda i: (0, i, 0)),
                  pl.BlockSpec((1, M), lambda i: (0, 0)),
                  pl.BlockSpec((M, OUT), lambda i: (0, 0)),
                  pl.BlockSpec((1, OUT), lambda i: (0, 0))],
        out_specs=pl.BlockSpec((BLK, OUT), lambda i: (i, 0)),
        out_shape=jax.ShapeDtypeStruct((N, OUT), jnp.float32),
    )(q, g, degp, b, wf, bf)


def kernel(x, edge_index, W1, b1, W2, b2, Wf, bf):
    E = edge_index.shape[1]
    pad = EPAD - E
    src = jnp.concatenate([edge_index[0],
                           jnp.zeros((pad,), jnp.int32)])
    dst = jnp.concatenate([edge_index[1],
                           N + (jnp.arange(pad, dtype=jnp.int32) % (NPAD - N))])
    idx = jnp.stack([src.reshape(-1, CHUNK), dst.reshape(-1, CHUNK)], axis=1)

    degp = _sc_degree(idx)
    degp = degp[:, :N, :]

    h1 = _tc_matmul(x, W1)
    g1 = _tc_scale(h1, degp)

    q1 = _sc_aggregate(g1, idx)[:, :N, :]
    g2 = _tc_combine_matmul_scale(q1, g1, degp, b1.reshape(1, HID), W2)

    q2 = _sc_aggregate(g2, idx)[:, :N, :]
    return _tc_final(q2, g2, degp, b2.reshape(1, HID), Wf, bf.reshape(1, OUT))


# speedup vs baseline: 28.1512x; 1.9092x over previous
import functools

import jax
import jax.numpy as jnp
from jax import lax
from jax.experimental import pallas as pl
from jax.experimental.pallas import tpu as pltpu
from jax.experimental.pallas import tpu_sc as plsc

N = 10000
NPAD = 10240
IN_DIM = 128
HID = 64
HHID = HID // 2
OUT = 16

NSC = 2
NTILE = 16
NW = NSC * NTILE
CHUNK = 128
EPT_CHUNKS = 80
EPT = CHUNK * EPT_CHUNKS
EPAD = EPT * NW
ROWS_PT = NPAD // NTILE
GROWS_PT = N // NTILE
NBUF = 8
DEPTH = 4

BLK = 1000
GRID = N // BLK


def _vsc_mesh():
    return plsc.VectorSubcoreMesh(core_axis_name="c", subcore_axis_name="s")


_SC_PARAMS = pltpu.CompilerParams(use_tc_tiling_on_sc=False)


def _sc_degree(idx):
    @functools.partial(
        pl.kernel,
        out_type=jax.ShapeDtypeStruct((NSC, NPAD, 16), jnp.float32),
        mesh=_vsc_mesh(),
        scratch_types=[
            pltpu.VMEM((1, CHUNK), jnp.int32),
            pltpu.VMEM((CHUNK, 16), jnp.float32),
            pltpu.VMEM_SHARED((NPAD, 16), jnp.float32),
        ],
        compiler_params=_SC_PARAMS,
    )
    def k(idx_hbm, out_hbm, didx, ones, acc):
        c = lax.axis_index("c")
        s = lax.axis_index("s")
        wid = c * NTILE + s

        @pl.loop(0, CHUNK)
        def _(i):
            ones[i, :] = jnp.zeros((16,), jnp.float32)

        @pl.loop(0, ROWS_PT // CHUNK)
        def _(r):
            pltpu.sync_copy(ones.at[:], acc.at[pl.ds(s * ROWS_PT + r * CHUNK, CHUNK)])

        @pl.loop(0, CHUNK)
        def _(i):
            ones[i, :] = jnp.ones((16,), jnp.float32)

        plsc.subcore_barrier()

        @pl.loop(0, EPT_CHUNKS)
        def _(ci):
            pltpu.sync_copy(idx_hbm.at[wid * EPT_CHUNKS + ci, 1], didx.at[0])
            pltpu.sync_copy(ones, acc.at[didx.at[0]], add=True)

        plsc.subcore_barrier()
        pltpu.sync_copy(acc.at[pl.ds(s * ROWS_PT, ROWS_PT)],
                        out_hbm.at[c, pl.ds(s * ROWS_PT, ROWS_PT)])

    return k(idx)


def _sc_aggregate(g, idx):
    @functools.partial(
        pl.kernel,
        out_type=jax.ShapeDtypeStruct((2, NSC, NPAD, HHID), jnp.float32),
        mesh=_vsc_mesh(),
        scratch_types=[
            pltpu.VMEM((EPT_CHUNKS, 2, CHUNK), jnp.int32),
            pltpu.VMEM((NBUF, CHUNK, HHID), jnp.float32),
            pltpu.VMEM((16, HHID), jnp.float32),
            pltpu.VMEM_SHARED((NPAD, HHID), jnp.float32),
            pltpu.VMEM_SHARED((N, HHID), jnp.float32),
            pltpu.SemaphoreType.DMA((NBUF,)),
            pltpu.SemaphoreType.DMA((NBUF,)),
        ],
        compiler_params=_SC_PARAMS,
    )
    def k(g_hbm, idx_hbm, out_hbm, idx_all, rows, zbuf, acc, gtab, sem_g, sem_s):
        c = lax.axis_index("c")
        s = lax.axis_index("s")
        wid = c * NTILE + s

        pltpu.sync_copy(idx_hbm.at[pl.ds(wid * EPT_CHUNKS, EPT_CHUNKS)], idx_all)

        @pl.loop(0, 16)
        def _(i):
            @pl.loop(0, HHID // 16)
            def _(j):
                zbuf[i, pl.ds(j * 16, 16)] = jnp.zeros((16,), jnp.float32)

        def start_gather(slot, ci):
            pltpu.async_copy(gtab.at[idx_all.at[ci, 0]], rows.at[slot],
                             sem_g.at[slot])

        def wait_gather(slot, ci):
            pltpu.make_async_copy(gtab.at[idx_all.at[ci, 0]], rows.at[slot],
                                  sem_g.at[slot]).wait()

        def start_scatter(slot, ci):
            pltpu.async_copy(rows.at[slot], acc.at[idx_all.at[ci, 1]],
                             sem_s.at[slot], add=True)

        def wait_scatter(slot, ci):
            pltpu.make_async_copy(rows.at[slot], acc.at[idx_all.at[ci, 1]],
                                  sem_s.at[slot]).wait()

        for h in (0, 1):
            pltpu.sync_copy(g_hbm.at[h, pl.ds(s * GROWS_PT, GROWS_PT)],
                            gtab.at[pl.ds(s * GROWS_PT, GROWS_PT)])

            @pl.loop(0, ROWS_PT // 16)
            def _(r):
                pltpu.sync_copy(zbuf, acc.at[pl.ds(s * ROWS_PT + r * 16, 16)])

            plsc.subcore_barrier()

            for b in range(DEPTH):
                start_gather(b, b)

            @pl.loop(0, EPT_CHUNKS, step=NBUF)
            def _(t):
                for b in range(NBUF):
                    ci = t + b
                    wait_gather(b, ci)
                    start_scatter(b, ci)
                    cg = ci + DEPTH
                    slot_g = (b + DEPTH) % NBUF

                    @pl.when(cg < EPT_CHUNKS)
                    def _():
                        @pl.when(cg >= NBUF)
                        def _():
                            wait_scatter(slot_g, ci)
                        start_gather(slot_g, cg)

            for b in range(NBUF):
                wait_scatter(b, EPT_CHUNKS - NBUF + b)

            plsc.subcore_barrier()
            pltpu.sync_copy(acc.at[pl.ds(s * ROWS_PT, ROWS_PT)],
                            out_hbm.at[h, c, pl.ds(s * ROWS_PT, ROWS_PT)])

    return k(g, idx)


def _tc_matmul(x, w):
    K, M = w.shape

    def body(x_ref, w_ref, o_ref):
        o_ref[...] = jnp.dot(x_ref[...], w_ref[...],
                             preferred_element_type=jnp.float32)

    return pl.pallas_call(
        body,
        grid=(GRID,),
        in_specs=[pl.BlockSpec((BLK, K), lambda i: (i, 0)),
                  pl.BlockSpec((K, M), lambda i: (0, 0))],
        out_specs=pl.BlockSpec((BLK, M), lambda i: (i, 0)),
        out_shape=jax.ShapeDtypeStruct((N, M), jnp.float32),
    )(x, w)


def _dis_of(d_ref):
    deg = d_ref[0, :, 0:1] + d_ref[1, :, 0:1] + 1.0
    return lax.rsqrt(deg)


_DEG_SPEC = pl.BlockSpec((NSC, BLK, 16), lambda i: (0, i, 0))
_GP_SPEC = pl.BlockSpec((2, BLK, HHID), lambda i: (0, i, 0))
_QP_SPEC = pl.BlockSpec((2, NSC, BLK, HHID), lambda i: (0, 0, i, 0))


def _split(o_ref, v):
    o_ref[0] = v[:, :HHID]
    o_ref[1] = v[:, HHID:]


def _tc_scale(h, degp):
    def body(h_ref, d_ref, o_ref):
        _split(o_ref, h_ref[...] * _dis_of(d_ref))

    return pl.pallas_call(
        body,
        grid=(GRID,),
        in_specs=[pl.BlockSpec((BLK, HID), lambda i: (i, 0)), _DEG_SPEC],
        out_specs=_GP_SPEC,
        out_shape=jax.ShapeDtypeStruct((2, N, HHID), jnp.float32),
    )(h, degp)


def _combine(q_ref, g_ref, d_ref, b_ref):
    dis = _dis_of(d_ref)
    t = jnp.concatenate(
        [dis * (q_ref[0, 0] + q_ref[0, 1] + g_ref[0]),
         dis * (q_ref[1, 0] + q_ref[1, 1] + g_ref[1])], axis=1) + b_ref[...]
    return jnp.maximum(t, 0.0)


def _tc_combine_matmul_scale(q, g, degp, b, w):
    def body(q_ref, g_ref, d_ref, b_ref, w_ref, o_ref):
        t = _combine(q_ref, g_ref, d_ref, b_ref)
        _split(o_ref, jnp.dot(t, w_ref[...],
                              preferred_element_type=jnp.float32) * _dis_of(d_ref))

    return pl.pallas_call(
        body,
        grid=(GRID,),
        in_specs=[_QP_SPEC, _GP_SPEC, _DEG_SPEC,
                  pl.BlockSpec((1, HID), lambda i: (0, 0)),
                  pl.BlockSpec((HID, HID), lambda i: (0, 0))],
        out_specs=_GP_SPEC,
        out_shape=jax.ShapeDtypeStruct((2, N, HHID), jnp.float32),
    )(q, g, degp, b, w)


def _tc_final(q, g, degp, b, wf, bf):
    def body(q_ref, g_ref, d_ref, b_ref, w_ref, bf_ref, o_ref):
        t = _combine(q_ref, g_ref, d_ref, b_ref)
        logits = jnp.dot(t, w_ref[...],
                         preferred_element_type=jnp.float32) + bf_ref[...]
        m = jnp.max(logits, axis=1, keepdims=True)
        lse = jnp.log(jnp.sum(jnp.exp(logits - m), axis=1, keepdims=True)) + m
        o_ref[...] = logits - lse

    return pl.pallas_call(
        body,
        grid=(GRID,),
        in_specs=[_QP_SPEC, _GP_SPEC, _DEG_SPEC,
                  pl.BlockSpec((1, HID), lambda i: (0, 0)),
                  pl.BlockSpec((HID, OUT), lambda i: (0, 0)),
                  pl.BlockSpec((1, OUT), lambda i: (0, 0))],
        out_specs=pl.BlockSpec((BLK, OUT), lambda i: (i, 0)),
        out_shape=jax.ShapeDtypeStruct((N, OUT), jnp.float32),
    )(q, g, degp, b, wf, bf)


def kernel(x, edge_index, W1, b1, W2, b2, Wf, bf):
    E = edge_index.shape[1]
    pad = EPAD - E
    src = jnp.concatenate([edge_index[0],
                           jnp.zeros((pad,), jnp.int32)])
    dst = jnp.concatenate([edge_index[1],
                           N + (jnp.arange(pad, dtype=jnp.int32) % (NPAD - N))])
    idx = jnp.stack([src.reshape(-1, CHUNK), dst.reshape(-1, CHUNK)], axis=1)

    degp = _sc_degree(idx)

    h1 = _tc_matmul(x, W1)
    g1 = _tc_scale(h1, degp)

    q1 = _sc_aggregate(g1, idx)
    g2 = _tc_combine_matmul_scale(q1, g1, degp, b1.reshape(1, HID), W2)

    q2 = _sc_aggregate(g2, idx)
    return _tc_final(q2, g2, degp, b2.reshape(1, HID), Wf, bf.reshape(1, OUT))


# speedup vs baseline: 31.8980x; 1.1331x over previous
import functools

import jax
import jax.numpy as jnp
from jax import lax
from jax.experimental import pallas as pl
from jax.experimental.pallas import tpu as pltpu
from jax.experimental.pallas import tpu_sc as plsc

N = 10000
NPAD = 10240
IN_DIM = 128
HID = 64
HHID = HID // 2
OUT = 16

NSC = 2
NTILE = 16
NW = NSC * NTILE
CHUNK = 128
EPT_CHUNKS = 80
EPT = CHUNK * EPT_CHUNKS
EPAD = EPT * NW
NCH = EPAD // CHUNK
ROWS_PT = NPAD // NTILE
GROWS_PT = N // NTILE
NBUF = 8
DEPTH = 4

BLK = 1000
GRID = N // BLK


def _vsc_mesh():
    return plsc.VectorSubcoreMesh(core_axis_name="c", subcore_axis_name="s")


_SC_PARAMS = pltpu.CompilerParams(use_tc_tiling_on_sc=False)


def _sc_degree(dst):
    @functools.partial(
        pl.kernel,
        out_type=jax.ShapeDtypeStruct((NSC, NPAD, 16), jnp.float32),
        mesh=_vsc_mesh(),
        scratch_types=[
            pltpu.VMEM((EPT_CHUNKS, CHUNK), jnp.int32),
            pltpu.VMEM((CHUNK, 16), jnp.float32),
            pltpu.VMEM_SHARED((NPAD, 16), jnp.float32),
            pltpu.SemaphoreType.DMA((NBUF,)),
        ],
        compiler_params=_SC_PARAMS,
    )
    def k(dst_hbm, out_hbm, idx_d, ones, acc, sem_s):
        c = lax.axis_index("c")
        s = lax.axis_index("s")
        wid = c * NTILE + s

        pltpu.sync_copy(dst_hbm.at[pl.ds(wid * EPT_CHUNKS, EPT_CHUNKS)], idx_d)

        @pl.loop(0, CHUNK)
        def _(i):
            ones[i, :] = jnp.zeros((16,), jnp.float32)

        @pl.loop(0, ROWS_PT // CHUNK)
        def _(r):
            pltpu.sync_copy(ones.at[:], acc.at[pl.ds(s * ROWS_PT + r * CHUNK, CHUNK)])

        @pl.loop(0, CHUNK)
        def _(i):
            ones[i, :] = jnp.ones((16,), jnp.float32)

        plsc.subcore_barrier()

        def start_scatter(slot, ci):
            pltpu.async_copy(ones, acc.at[idx_d.at[ci]], sem_s.at[slot],
                             add=True)

        def wait_scatter(slot, ci):
            pltpu.make_async_copy(ones, acc.at[idx_d.at[ci]],
                                  sem_s.at[slot]).wait()

        @pl.loop(0, EPT_CHUNKS, step=NBUF)
        def _(t):
            for b in range(NBUF):
                ci = t + b

                @pl.when(ci >= NBUF)
                def _():
                    wait_scatter(b, ci)
                start_scatter(b, ci)

        for b in range(NBUF):
            wait_scatter(b, EPT_CHUNKS - NBUF + b)

        plsc.subcore_barrier()
        pltpu.sync_copy(acc.at[pl.ds(s * ROWS_PT, ROWS_PT)],
                        out_hbm.at[c, pl.ds(s * ROWS_PT, ROWS_PT)])

    return k(dst)


def _sc_aggregate(g, src, dst):
    @functools.partial(
        pl.kernel,
        out_type=jax.ShapeDtypeStruct((2, NSC, NPAD, HHID), jnp.float32),
        mesh=_vsc_mesh(),
        scratch_types=[
            pltpu.VMEM((EPT_CHUNKS, CHUNK), jnp.int32),
            pltpu.VMEM((EPT_CHUNKS, CHUNK), jnp.int32),
            pltpu.VMEM((NBUF, CHUNK, HHID), jnp.float32),
            pltpu.VMEM((16, HHID), jnp.float32),
            pltpu.VMEM_SHARED((NPAD, HHID), jnp.float32),
            pltpu.VMEM_SHARED((N, HHID), jnp.float32),
            pltpu.SemaphoreType.DMA((NBUF,)),
            pltpu.SemaphoreType.DMA((NBUF,)),
        ],
        compiler_params=_SC_PARAMS,
    )
    def k(g_hbm, src_hbm, dst_hbm, out_hbm, idx_s, idx_d, rows, zbuf, acc,
          gtab, sem_g, sem_s):
        c = lax.axis_index("c")
        s = lax.axis_index("s")
        wid = c * NTILE + s

        pltpu.sync_copy(src_hbm.at[pl.ds(wid * EPT_CHUNKS, EPT_CHUNKS)], idx_s)
        pltpu.sync_copy(dst_hbm.at[pl.ds(wid * EPT_CHUNKS, EPT_CHUNKS)], idx_d)

        @pl.loop(0, 16)
        def _(i):
            @pl.loop(0, HHID // 16)
            def _(j):
                zbuf[i, pl.ds(j * 16, 16)] = jnp.zeros((16,), jnp.float32)

        def start_gather(slot, ci):
            pltpu.async_copy(gtab.at[idx_s.at[ci]], rows.at[slot],
                             sem_g.at[slot])

        def wait_gather(slot, ci):
            pltpu.make_async_copy(gtab.at[idx_s.at[ci]], rows.at[slot],
                                  sem_g.at[slot]).wait()

        def start_scatter(slot, ci):
            pltpu.async_copy(rows.at[slot], acc.at[idx_d.at[ci]],
                             sem_s.at[slot], add=True)

        def wait_scatter(slot, ci):
            pltpu.make_async_copy(rows.at[slot], acc.at[idx_d.at[ci]],
                                  sem_s.at[slot]).wait()

        for h in (0, 1):
            pltpu.sync_copy(g_hbm.at[h, pl.ds(s * GROWS_PT, GROWS_PT)],
                            gtab.at[pl.ds(s * GROWS_PT, GROWS_PT)])

            @pl.loop(0, ROWS_PT // 16)
            def _(r):
                pltpu.sync_copy(zbuf, acc.at[pl.ds(s * ROWS_PT + r * 16, 16)])

            plsc.subcore_barrier()

            for b in range(DEPTH):
                start_gather(b, b)

            @pl.loop(0, EPT_CHUNKS, step=NBUF)
            def _(t):
                for b in range(NBUF):
                    ci = t + b
                    wait_gather(b, ci)
                    start_scatter(b, ci)
                    cg = ci + DEPTH
                    slot_g = (b + DEPTH) % NBUF

                    @pl.when(cg < EPT_CHUNKS)
                    def _():
                        @pl.when(cg >= NBUF)
                        def _():
                            wait_scatter(slot_g, ci)
                        start_gather(slot_g, cg)

            for b in range(NBUF):
                wait_scatter(b, EPT_CHUNKS - NBUF + b)

            plsc.subcore_barrier()
            pltpu.sync_copy(acc.at[pl.ds(s * ROWS_PT, ROWS_PT)],
                            out_hbm.at[h, c, pl.ds(s * ROWS_PT, ROWS_PT)])

    return k(g, src, dst)


def _tc_matmul(x, w):
    K, M = w.shape

    def body(x_ref, w_ref, o_ref):
        o_ref[...] = jnp.dot(x_ref[...], w_ref[...],
                             preferred_element_type=jnp.float32)

    return pl.pallas_call(
        body,
        grid=(GRID,),
        in_specs=[pl.BlockSpec((BLK, K), lambda i: (i, 0)),
                  pl.BlockSpec((K, M), lambda i: (0, 0))],
        out_specs=pl.BlockSpec((BLK, M), lambda i: (i, 0)),
        out_shape=jax.ShapeDtypeStruct((N, M), jnp.float32),
    )(x, w)


def _dis_of(d_ref):
    deg = d_ref[0, :, 0:1] + d_ref[1, :, 0:1] + 1.0
    return lax.rsqrt(deg)


_DEG_SPEC = pl.BlockSpec((NSC, BLK, 16), lambda i: (0, i, 0))
_GP_SPEC = pl.BlockSpec((2, BLK, HHID), lambda i: (0, i, 0))
_QP_SPEC = pl.BlockSpec((2, NSC, BLK, HHID), lambda i: (0, 0, i, 0))
_BH_SPEC = pl.BlockSpec((2, 1, HHID), lambda i: (0, 0, 0))
_WH_SPEC = lambda m2: pl.BlockSpec((2, HHID, m2), lambda i: (0, 0, 0))


def _tc_scale(h, degp):
    def body(h_ref, d_ref, o_ref):
        dis = _dis_of(d_ref)
        o_ref[0] = h_ref[:, :HHID] * dis
        o_ref[1] = h_ref[:, HHID:] * dis

    return pl.pallas_call(
        body,
        grid=(GRID,),
        in_specs=[pl.BlockSpec((BLK, HID), lambda i: (i, 0)), _DEG_SPEC],
        out_specs=_GP_SPEC,
        out_shape=jax.ShapeDtypeStruct((2, N, HHID), jnp.float32),
    )(h, degp)


def _relu_halves(q_ref, g_ref, d_ref, b_ref):
    dis = _dis_of(d_ref)
    return tuple(
        jnp.maximum(dis * (q_ref[h, 0] + q_ref[h, 1] + g_ref[h]) + b_ref[h],
                    0.0)
        for h in (0, 1))


def _tc_combine_matmul_scale(q, g, degp, bh, wh):
    def body(q_ref, g_ref, d_ref, b_ref, w_ref, o_ref):
        t0, t1 = _relu_halves(q_ref, g_ref, d_ref, b_ref)
        mm = (jnp.dot(t0, w_ref[0], preferred_element_type=jnp.float32) +
              jnp.dot(t1, w_ref[1], preferred_element_type=jnp.float32))
        mm = mm * _dis_of(d_ref)
        o_ref[0] = mm[:, :HHID]
        o_ref[1] = mm[:, HHID:]

    return pl.pallas_call(
        body,
        grid=(GRID,),
        in_specs=[_QP_SPEC, _GP_SPEC, _DEG_SPEC, _BH_SPEC, _WH_SPEC(HID)],
        out_specs=_GP_SPEC,
        out_shape=jax.ShapeDtypeStruct((2, N, HHID), jnp.float32),
    )(q, g, degp, bh, wh)


def _tc_final(q, g, degp, bh, wfh, bf):
    def body(q_ref, g_ref, d_ref, b_ref, w_ref, bf_ref, o_ref):
        t0, t1 = _relu_halves(q_ref, g_ref, d_ref, b_ref)
        logits = (jnp.dot(t0, w_ref[0], preferred_element_type=jnp.float32) +
                  jnp.dot(t1, w_ref[1], preferred_element_type=jnp.float32)
                  ) + bf_ref[...]
        m = jnp.max(logits, axis=1, keepdims=True)
        lse = jnp.log(jnp.sum(jnp.exp(logits - m), axis=1, keepdims=True)) + m
        o_ref[...] = logits - lse

    return pl.pallas_call(
        body,
        grid=(GRID,),
        in_specs=[_QP_SPEC, _GP_SPEC, _DEG_SPEC, _BH_SPEC, _WH_SPEC(OUT),
                  pl.BlockSpec((1, OUT), lambda i: (0, 0))],
        out_specs=pl.BlockSpec((BLK, OUT), lambda i: (i, 0)),
        out_shape=jax.ShapeDtypeStruct((N, OUT), jnp.float32),
    )(q, g, degp, bh, wfh, bf)


def _halves(w):
    return jnp.stack([w[:HHID], w[HHID:]])


def kernel(x, edge_index, W1, b1, W2, b2, Wf, bf):
    E = edge_index.shape[1]
    pad = EPAD - E
    src = jnp.concatenate([edge_index[0],
                           jnp.zeros((pad,), jnp.int32)]).reshape(NCH, CHUNK)
    dst = jnp.concatenate(
        [edge_index[1],
         N + (jnp.arange(pad, dtype=jnp.int32) % (NPAD - N))]).reshape(NCH, CHUNK)

    degp = _sc_degree(dst)

    h1 = _tc_matmul(x, W1)
    g1 = _tc_scale(h1, degp)

    b1h = b1.reshape(2, 1, HHID)
    b2h = b2.reshape(2, 1, HHID)

    q1 = _sc_aggregate(g1, src, dst)
    g2 = _tc_combine_matmul_scale(q1, g1, degp, b1h, _halves(W2))

    q2 = _sc_aggregate(g2, src, dst)
    return _tc_final(q2, g2, degp, b2h, _halves(Wf), bf.reshape(1, OUT))


# speedup vs baseline: 41.2737x; 1.2939x over previous
import functools

import jax
import jax.numpy as jnp
from jax import lax
from jax.experimental import pallas as pl
from jax.experimental.pallas import tpu as pltpu
from jax.experimental.pallas import tpu_sc as plsc

N = 10000
NPAD = 10240
IN_DIM = 128
HID = 64
HHID = HID // 2
OUT = 16
LANE = 128

NSC = 2
NTILE = 16
NW = NSC * NTILE
CHUNK = 128
EPT = 10240
EPT_CHUNKS = EPT // CHUNK
EPAD = EPT * NW
NCH = EPAD // CHUNK
CPT = NCH // NTILE
ROWS_PT = NPAD // NTILE
GROWS_PT = N // NTILE
NBUF = 8
DEPTH = 4

BLK = 1000
GRID = N // BLK


def _vsc_mesh():
    return plsc.VectorSubcoreMesh(core_axis_name="c", subcore_axis_name="s")


_SC_PARAMS = pltpu.CompilerParams(use_tc_tiling_on_sc=False)


def _sc_degree(dst):
    @functools.partial(
        pl.kernel,
        out_type=jax.ShapeDtypeStruct((NSC, NPAD, 16), jnp.float32),
        mesh=_vsc_mesh(),
        scratch_types=[
            pltpu.VMEM((EPT_CHUNKS, CHUNK), jnp.int32),
            pltpu.VMEM((CHUNK, 16), jnp.float32),
            pltpu.VMEM_SHARED((NPAD, 16), jnp.float32),
            pltpu.SemaphoreType.DMA((NBUF,)),
        ],
        compiler_params=_SC_PARAMS,
    )
    def k(dst_hbm, out_hbm, idx_d, ones, acc, sem_s):
        c = lax.axis_index("c")
        s = lax.axis_index("s")
        wid = c * NTILE + s

        pltpu.sync_copy(dst_hbm.at[pl.ds(wid * EPT_CHUNKS, EPT_CHUNKS)], idx_d)

        @pl.loop(0, CHUNK)
        def _(i):
            ones[i, :] = jnp.zeros((16,), jnp.float32)

        @pl.loop(0, ROWS_PT // CHUNK)
        def _(r):
            pltpu.sync_copy(ones.at[:], acc.at[pl.ds(s * ROWS_PT + r * CHUNK, CHUNK)])

        @pl.loop(0, CHUNK)
        def _(i):
            ones[i, :] = jnp.ones((16,), jnp.float32)

        plsc.subcore_barrier()

        def start_scatter(slot, ci):
            pltpu.async_copy(ones, acc.at[idx_d.at[ci]], sem_s.at[slot],
                             add=True)

        def wait_scatter(slot, ci):
            pltpu.make_async_copy(ones, acc.at[idx_d.at[ci]],
                                  sem_s.at[slot]).wait()

        @pl.loop(0, EPT_CHUNKS, step=NBUF)
        def _(t):
            for b in range(NBUF):
                ci = t + b

                @pl.when(ci >= NBUF)
                def _():
                    wait_scatter(b, ci)
                start_scatter(b, ci)

        for b in range(NBUF):
            wait_scatter(b, EPT_CHUNKS - NBUF + b)

        plsc.subcore_barrier()
        pltpu.sync_copy(acc.at[pl.ds(s * ROWS_PT, ROWS_PT)],
                        out_hbm.at[c, pl.ds(s * ROWS_PT, ROWS_PT)])

    return k(dst)


def _sc_aggregate(g, src, dst):
    @functools.partial(
        pl.kernel,
        out_type=jax.ShapeDtypeStruct((NPAD, LANE), jnp.float32),
        mesh=_vsc_mesh(),
        scratch_types=[
            pltpu.VMEM((CPT, CHUNK), jnp.int32),
            pltpu.VMEM((CPT, CHUNK), jnp.int32),
            pltpu.VMEM((NBUF, CHUNK, HHID), jnp.float32),
            pltpu.VMEM((16, HHID), jnp.float32),
            pltpu.VMEM_SHARED((NPAD, HHID), jnp.float32),
            pltpu.VMEM_SHARED((N, HHID), jnp.float32),
            pltpu.SemaphoreType.DMA((NBUF,)),
            pltpu.SemaphoreType.DMA((NBUF,)),
        ],
        compiler_params=_SC_PARAMS,
    )
    def k(g_hbm, src_hbm, dst_hbm, out_hbm, idx_s, idx_d, rows, zbuf, acc,
          gtab, sem_g, sem_s):
        c = lax.axis_index("c")
        s = lax.axis_index("s")
        col = c * HHID

        pltpu.sync_copy(src_hbm.at[pl.ds(s * CPT, CPT)], idx_s)
        pltpu.sync_copy(dst_hbm.at[pl.ds(s * CPT, CPT)], idx_d)

        pltpu.sync_copy(g_hbm.at[pl.ds(s * GROWS_PT, GROWS_PT),
                                 pl.ds(col, HHID)],
                        gtab.at[pl.ds(s * GROWS_PT, GROWS_PT)])

        @pl.loop(0, 16)
        def _(i):
            @pl.loop(0, HHID // 16)
            def _(j):
                zbuf[i, pl.ds(j * 16, 16)] = jnp.zeros((16,), jnp.float32)

        @pl.when(s < NTILE - 1)
        def _():
            pltpu.sync_copy(g_hbm.at[pl.ds(s * ROWS_PT, ROWS_PT),
                                     pl.ds(col, HHID)],
                            acc.at[pl.ds(s * ROWS_PT, ROWS_PT)])

        @pl.when(s == NTILE - 1)
        def _():
            last = (NTILE - 1) * ROWS_PT
            pltpu.sync_copy(g_hbm.at[pl.ds(last, N - last), pl.ds(col, HHID)],
                            acc.at[pl.ds(last, N - last)])

            @pl.loop(N, NPAD, step=16)
            def _(r):
                pltpu.sync_copy(zbuf, acc.at[pl.ds(r, 16)])

        plsc.subcore_barrier()

        def start_gather(slot, ci):
            pltpu.async_copy(gtab.at[idx_s.at[ci]], rows.at[slot],
                             sem_g.at[slot])

        def wait_gather(slot, ci):
            pltpu.make_async_copy(gtab.at[idx_s.at[ci]], rows.at[slot],
                                  sem_g.at[slot]).wait()

        def start_scatter(slot, ci):
            pltpu.async_copy(rows.at[slot], acc.at[idx_d.at[ci]],
                             sem_s.at[slot], add=True)

        def wait_scatter(slot, ci):
            pltpu.make_async_copy(rows.at[slot], acc.at[idx_d.at[ci]],
                                  sem_s.at[slot]).wait()

        for b in range(DEPTH):
            start_gather(b, b)

        @pl.loop(0, CPT, step=NBUF)
        def _(t):
            for b in range(NBUF):
                ci = t + b
                wait_gather(b, ci)
                start_scatter(b, ci)
                cg = ci + DEPTH
                slot_g = (b + DEPTH) % NBUF

                @pl.when(cg < CPT)
                def _():
                    @pl.when(cg >= NBUF)
                    def _():
                        wait_scatter(slot_g, ci)
                    start_gather(slot_g, cg)

        for b in range(NBUF):
            wait_scatter(b, CPT - NBUF + b)

        plsc.subcore_barrier()
        pltpu.sync_copy(acc.at[pl.ds(s * ROWS_PT, ROWS_PT)],
                        out_hbm.at[pl.ds(s * ROWS_PT, ROWS_PT),
                                   pl.ds(col, HHID)])

    return k(g, src, dst)


def _tc_matmul(x, w):
    K, M = w.shape

    def body(x_ref, w_ref, o_ref):
        o_ref[...] = jnp.dot(x_ref[...], w_ref[...],
                             preferred_element_type=jnp.float32)

    return pl.pallas_call(
        body,
        grid=(GRID,),
        in_specs=[pl.BlockSpec((BLK, K), lambda i: (i, 0)),
                  pl.BlockSpec((K, M), lambda i: (0, 0))],
        out_specs=pl.BlockSpec((BLK, M), lambda i: (i, 0)),
        out_shape=jax.ShapeDtypeStruct((N, M), jnp.float32),
    )(x, w)


def _dis_of(d_ref):
    deg = d_ref[0, :, 0:1] + d_ref[1, :, 0:1] + 1.0
    return lax.rsqrt(deg)


_DEG_SPEC = pl.BlockSpec((NSC, BLK, 16), lambda i: (0, i, 0))
_LANE_SPEC = pl.BlockSpec((BLK, LANE), lambda i: (i, 0))


def _tc_scale(h, degp):
    def body(h_ref, d_ref, o_ref):
        o_ref[:, :HID] = h_ref[...] * _dis_of(d_ref)

    return pl.pallas_call(
        body,
        grid=(GRID,),
        in_specs=[pl.BlockSpec((BLK, HID), lambda i: (i, 0)), _DEG_SPEC],
        out_specs=_LANE_SPEC,
        out_shape=jax.ShapeDtypeStruct((N, LANE), jnp.float32),
    )(h, degp)


def _tc_combine_matmul_scale(u, degp, b, w):
    def body(u_ref, d_ref, b_ref, w_ref, o_ref):
        dis = _dis_of(d_ref)
        t = jnp.maximum(dis * u_ref[:, :HID] + b_ref[...], 0.0)
        o_ref[:, :HID] = jnp.dot(t, w_ref[...],
                                 preferred_element_type=jnp.float32) * dis

    return pl.pallas_call(
        body,
        grid=(GRID,),
        in_specs=[_LANE_SPEC, _DEG_SPEC,
                  pl.BlockSpec((1, HID), lambda i: (0, 0)),
                  pl.BlockSpec((HID, HID), lambda i: (0, 0))],
        out_specs=_LANE_SPEC,
        out_shape=jax.ShapeDtypeStruct((N, LANE), jnp.float32),
    )(u, degp, b, w)


def _tc_final(u, degp, b, wf, bf):
    def body(u_ref, d_ref, b_ref, w_ref, bf_ref, o_ref):
        dis = _dis_of(d_ref)
        t = jnp.maximum(dis * u_ref[:, :HID] + b_ref[...], 0.0)
        logits = jnp.dot(t, w_ref[...],
                         preferred_element_type=jnp.float32) + bf_ref[...]
        m = jnp.max(logits, axis=1, keepdims=True)
        lse = jnp.log(jnp.sum(jnp.exp(logits - m), axis=1, keepdims=True)) + m
        o_ref[...] = logits - lse

    return pl.pallas_call(
        body,
        grid=(GRID,),
        in_specs=[_LANE_SPEC, _DEG_SPEC,
                  pl.BlockSpec((1, HID), lambda i: (0, 0)),
                  pl.BlockSpec((HID, OUT), lambda i: (0, 0)),
                  pl.BlockSpec((1, OUT), lambda i: (0, 0))],
        out_specs=pl.BlockSpec((BLK, OUT), lambda i: (i, 0)),
        out_shape=jax.ShapeDtypeStruct((N, OUT), jnp.float32),
    )(u, degp, b, wf, bf)


def kernel(x, edge_index, W1, b1, W2, b2, Wf, bf):
    E = edge_index.shape[1]
    pad = EPAD - E
    src = jnp.concatenate([edge_index[0],
                           jnp.zeros((pad,), jnp.int32)]).reshape(NCH, CHUNK)
    dst = jnp.concatenate(
        [edge_index[1],
         N + (jnp.arange(pad, dtype=jnp.int32) % (NPAD - N))]).reshape(NCH, CHUNK)

    degp = _sc_degree(dst)

    h1 = _tc_matmul(x, W1)
    g1 = _tc_scale(h1, degp)

    u1 = _sc_aggregate(g1, src, dst)
    g2 = _tc_combine_matmul_scale(u1, degp, b1.reshape(1, HID), W2)

    u2 = _sc_aggregate(g2, src, dst)
    return _tc_final(u2, degp, b2.reshape(1, HID), Wf, bf.reshape(1, OUT))


# speedup vs baseline: 43.0777x; 1.0437x over previous
import functools

import jax
import jax.numpy as jnp
from jax import lax
from jax.experimental import pallas as pl
from jax.experimental.pallas import tpu as pltpu
from jax.experimental.pallas import tpu_sc as plsc

N = 10000
NPAD = 10240
IN_DIM = 128
HID = 64
HHID = HID // 2
OUT = 16
LANE = 128

NSC = 2
NTILE = 16
NW = NSC * NTILE
CHUNK = 128
E = 320000
ECH = E // CHUNK
EPT_CHUNKS = 80
EPAD = EPT_CHUNKS * CHUNK * NW
NCH = EPAD // CHUNK
PCH = NCH - ECH
CPT = NCH // NTILE
ROWS_PT = NPAD // NTILE
GROWS_PT = N // NTILE
NBUF = 8
DEPTH = 4

BLK = 1000
GRID = N // BLK


def _vsc_mesh():
    return plsc.VectorSubcoreMesh(core_axis_name="c", subcore_axis_name="s")


_SC_PARAMS = pltpu.CompilerParams(use_tc_tiling_on_sc=False)


def _sc_degree(em, dstpad):
    @functools.partial(
        pl.kernel,
        out_type=jax.ShapeDtypeStruct((NPAD, 32), jnp.float32),
        mesh=_vsc_mesh(),
        scratch_types=[
            pltpu.VMEM((EPT_CHUNKS, CHUNK), jnp.int32),
            pltpu.VMEM((CHUNK, 16), jnp.float32),
            pltpu.VMEM_SHARED((NPAD, 16), jnp.float32),
            pltpu.SemaphoreType.DMA((NBUF,)),
        ],
        compiler_params=_SC_PARAMS,
    )
    def k(em_hbm, pad_hbm, out_hbm, idx_d, ones, acc, sem_s):
        c = lax.axis_index("c")
        s = lax.axis_index("s")
        wid = c * NTILE + s

        @pl.when(wid < NW - 1)
        def _():
            pltpu.sync_copy(em_hbm.at[1, pl.ds(wid * EPT_CHUNKS, EPT_CHUNKS)],
                            idx_d)

        @pl.when(wid == NW - 1)
        def _():
            main = ECH - (NW - 1) * EPT_CHUNKS
            pltpu.sync_copy(em_hbm.at[1, pl.ds((NW - 1) * EPT_CHUNKS, main)],
                            idx_d.at[pl.ds(0, main)])
            pltpu.sync_copy(pad_hbm, idx_d.at[pl.ds(main, PCH)])

        @pl.loop(0, CHUNK)
        def _(i):
            ones[i, :] = jnp.zeros((16,), jnp.float32)

        @pl.loop(0, ROWS_PT // CHUNK)
        def _(r):
            pltpu.sync_copy(ones.at[:], acc.at[pl.ds(s * ROWS_PT + r * CHUNK, CHUNK)])

        @pl.loop(0, CHUNK)
        def _(i):
            ones[i, :] = jnp.ones((16,), jnp.float32)

        plsc.subcore_barrier()

        def start_scatter(slot, ci):
            pltpu.async_copy(ones, acc.at[idx_d.at[ci]], sem_s.at[slot],
                             add=True)

        def wait_scatter(slot, ci):
            pltpu.make_async_copy(ones, acc.at[idx_d.at[ci]],
                                  sem_s.at[slot]).wait()

        @pl.loop(0, EPT_CHUNKS, step=NBUF)
        def _(t):
            for b in range(NBUF):
                ci = t + b

                @pl.when(ci >= NBUF)
                def _():
                    wait_scatter(b, ci)
                start_scatter(b, ci)

        for b in range(NBUF):
            wait_scatter(b, EPT_CHUNKS - NBUF + b)

        plsc.subcore_barrier()
        pltpu.sync_copy(acc.at[pl.ds(s * ROWS_PT, ROWS_PT)],
                        out_hbm.at[pl.ds(s * ROWS_PT, ROWS_PT),
                                   pl.ds(16 * c, 16)])

    return k(em, dstpad)


def _sc_aggregate(g, em, srcpad, dstpad):
    @functools.partial(
        pl.kernel,
        out_type=jax.ShapeDtypeStruct((NPAD, LANE), jnp.float32),
        mesh=_vsc_mesh(),
        scratch_types=[
            pltpu.VMEM((CPT, CHUNK), jnp.int32),
            pltpu.VMEM((CPT, CHUNK), jnp.int32),
            pltpu.VMEM((NBUF, CHUNK, HHID), jnp.float32),
            pltpu.VMEM((16, HHID), jnp.float32),
            pltpu.VMEM((ROWS_PT, 16), jnp.float32),
            pltpu.VMEM_SHARED((NPAD, HHID), jnp.float32),
            pltpu.VMEM_SHARED((N, HHID), jnp.float32),
            pltpu.SemaphoreType.DMA((NBUF,)),
            pltpu.SemaphoreType.DMA((NBUF,)),
        ],
        compiler_params=_SC_PARAMS,
    )
    def k(g_hbm, em_hbm, spad_hbm, dpad_hbm, out_hbm, idx_s, idx_d, rows,
          zbuf, dbuf, acc, gtab, sem_g, sem_s):
        c = lax.axis_index("c")
        s = lax.axis_index("s")
        col = c * HHID

        @pl.when(s < NTILE - 1)
        def _():
            pltpu.sync_copy(em_hbm.at[0, pl.ds(s * CPT, CPT)], idx_s)
            pltpu.sync_copy(em_hbm.at[1, pl.ds(s * CPT, CPT)], idx_d)

        @pl.when(s == NTILE - 1)
        def _():
            main = ECH - (NTILE - 1) * CPT
            base = (NTILE - 1) * CPT
            pltpu.sync_copy(em_hbm.at[0, pl.ds(base, main)],
                            idx_s.at[pl.ds(0, main)])
            pltpu.sync_copy(em_hbm.at[1, pl.ds(base, main)],
                            idx_d.at[pl.ds(0, main)])
            pltpu.sync_copy(spad_hbm, idx_s.at[pl.ds(main, PCH)])
            pltpu.sync_copy(dpad_hbm, idx_d.at[pl.ds(main, PCH)])

        pltpu.sync_copy(g_hbm.at[pl.ds(s * GROWS_PT, GROWS_PT),
                                 pl.ds(col, HHID)],
                        gtab.at[pl.ds(s * GROWS_PT, GROWS_PT)])

        @pl.loop(0, 16)
        def _(i):
            @pl.loop(0, HHID // 16)
            def _(j):
                zbuf[i, pl.ds(j * 16, 16)] = jnp.zeros((16,), jnp.float32)

        @pl.when(s < NTILE - 1)
        def _():
            pltpu.sync_copy(g_hbm.at[pl.ds(s * ROWS_PT, ROWS_PT),
                                     pl.ds(col, HHID)],
                            acc.at[pl.ds(s * ROWS_PT, ROWS_PT)])

        @pl.when(s == NTILE - 1)
        def _():
            last = (NTILE - 1) * ROWS_PT
            pltpu.sync_copy(g_hbm.at[pl.ds(last, N - last), pl.ds(col, HHID)],
                            acc.at[pl.ds(last, N - last)])

            @pl.loop(N, NPAD, step=16)
            def _(r):
                pltpu.sync_copy(zbuf, acc.at[pl.ds(r, 16)])

        @pl.when(jnp.logical_and(c == 0, s < NTILE - 1))
        def _():
            pltpu.sync_copy(g_hbm.at[pl.ds(s * ROWS_PT, ROWS_PT),
                                     pl.ds(HID, 16)], dbuf)
            pltpu.sync_copy(dbuf, out_hbm.at[pl.ds(s * ROWS_PT, ROWS_PT),
                                             pl.ds(HID, 16)])

        @pl.when(jnp.logical_and(c == 0, s == NTILE - 1))
        def _():
            last = (NTILE - 1) * ROWS_PT
            pltpu.sync_copy(g_hbm.at[pl.ds(last, N - last), pl.ds(HID, 16)],
                            dbuf.at[pl.ds(0, N - last)])
            pltpu.sync_copy(dbuf.at[pl.ds(0, N - last)],
                            out_hbm.at[pl.ds(last, N - last), pl.ds(HID, 16)])

        plsc.subcore_barrier()

        def start_gather(slot, ci):
            pltpu.async_copy(gtab.at[idx_s.at[ci]], rows.at[slot],
                             sem_g.at[slot])

        def wait_gather(slot, ci):
            pltpu.make_async_copy(gtab.at[idx_s.at[ci]], rows.at[slot],
                                  sem_g.at[slot]).wait()

        def start_scatter(slot, ci):
            pltpu.async_copy(rows.at[slot], acc.at[idx_d.at[ci]],
                             sem_s.at[slot], add=True)

        def wait_scatter(slot, ci):
            pltpu.make_async_copy(rows.at[slot], acc.at[idx_d.at[ci]],
                                  sem_s.at[slot]).wait()

        for b in range(DEPTH):
            start_gather(b, b)

        @pl.loop(0, CPT, step=NBUF)
        def _(t):
            for b in range(NBUF):
                ci = t + b
                wait_gather(b, ci)
                start_scatter(b, ci)
                cg = ci + DEPTH
                slot_g = (b + DEPTH) % NBUF

                @pl.when(cg < CPT)
                def _():
                    @pl.when(cg >= NBUF)
                    def _():
                        wait_scatter(slot_g, ci)
                    start_gather(slot_g, cg)

        for b in range(NBUF):
            wait_scatter(b, CPT - NBUF + b)

        plsc.subcore_barrier()
        pltpu.sync_copy(acc.at[pl.ds(s * ROWS_PT, ROWS_PT)],
                        out_hbm.at[pl.ds(s * ROWS_PT, ROWS_PT),
                                   pl.ds(col, HHID)])

    return k(g, em, srcpad, dstpad)


def _tc_matmul(x, w):
    K, M = w.shape

    def body(x_ref, w_ref, o_ref):
        o_ref[...] = jnp.dot(x_ref[...], w_ref[...],
                             preferred_element_type=jnp.float32)

    return pl.pallas_call(
        body,
        grid=(GRID,),
        in_specs=[pl.BlockSpec((BLK, K), lambda i: (i, 0)),
                  pl.BlockSpec((K, M), lambda i: (0, 0))],
        out_specs=pl.BlockSpec((BLK, M), lambda i: (i, 0)),
        out_shape=jax.ShapeDtypeStruct((N, M), jnp.float32),
    )(x, w)


_LANE_SPEC = pl.BlockSpec((BLK, LANE), lambda i: (i, 0))


def _tc_scale(h, degp):
    def body(h_ref, d_ref, o_ref):
        deg = d_ref[:, 0:1] + d_ref[:, 16:17] + 1.0
        dis = lax.rsqrt(deg)
        o_ref[:, :HID] = h_ref[...] * dis
        o_ref[:, HID:HID + 1] = dis

    return pl.pallas_call(
        body,
        grid=(GRID,),
        in_specs=[pl.BlockSpec((BLK, HID), lambda i: (i, 0)),
                  pl.BlockSpec((BLK, 32), lambda i: (i, 0))],
        out_specs=_LANE_SPEC,
        out_shape=jax.ShapeDtypeStruct((N, LANE), jnp.float32),
    )(h, degp)


def _tc_combine_matmul_scale(u, b, w):
    def body(u_ref, b_ref, w_ref, o_ref):
        dis = u_ref[:, HID:HID + 1]
        t = jnp.maximum(dis * u_ref[:, :HID] + b_ref[...], 0.0)
        o_ref[:, :HID] = jnp.dot(t, w_ref[...],
                                 preferred_element_type=jnp.float32) * dis
        o_ref[:, HID:HID + 1] = dis

    return pl.pallas_call(
        body,
        grid=(GRID,),
        in_specs=[_LANE_SPEC,
                  pl.BlockSpec((1, HID), lambda i: (0, 0)),
                  pl.BlockSpec((HID, HID), lambda i: (0, 0))],
        out_specs=_LANE_SPEC,
        out_shape=jax.ShapeDtypeStruct((N, LANE), jnp.float32),
    )(u, b, w)


def _tc_final(u, b, wf, bf):
    def body(u_ref, b_ref, w_ref, bf_ref, o_ref):
        dis = u_ref[:, HID:HID + 1]
        t = jnp.maximum(dis * u_ref[:, :HID] + b_ref[...], 0.0)
        logits = jnp.dot(t, w_ref[...],
                         preferred_element_type=jnp.float32) + bf_ref[...]
        m = jnp.max(logits, axis=1, keepdims=True)
        lse = jnp.log(jnp.sum(jnp.exp(logits - m), axis=1, keepdims=True)) + m
        o_ref[...] = logits - lse

    return pl.pallas_call(
        body,
        grid=(GRID,),
        in_specs=[_LANE_SPEC,
                  pl.BlockSpec((1, HID), lambda i: (0, 0)),
                  pl.BlockSpec((HID, OUT), lambda i: (0, 0)),
                  pl.BlockSpec((1, OUT), lambda i: (0, 0))],
        out_specs=pl.BlockSpec((BLK, OUT), lambda i: (i, 0)),
        out_shape=jax.ShapeDtypeStruct((N, OUT), jnp.float32),
    )(u, b, wf, bf)


def kernel(x, edge_index, W1, b1, W2, b2, Wf, bf):
    em = edge_index.reshape(2, ECH, CHUNK)
    srcpad = jnp.zeros((PCH, CHUNK), jnp.int32)
    dstpad = (N + (jnp.arange(PCH * CHUNK, dtype=jnp.int32) % (NPAD - N))
              ).reshape(PCH, CHUNK)

    degp = _sc_degree(em, dstpad)

    h1 = _tc_matmul(x, W1)
    g1 = _tc_scale(h1, degp)

    u1 = _sc_aggregate(g1, em, srcpad, dstpad)
    g2 = _tc_combine_matmul_scale(u1, b1.reshape(1, HID), W2)

    u2 = _sc_aggregate(g2, em, srcpad, dstpad)
    return _tc_final(u2, b2.reshape(1, HID), Wf, bf.reshape(1, OUT))


# speedup vs baseline: 44.0147x; 1.0218x over previous
import functools

import jax
import jax.numpy as jnp
from jax import lax
from jax.experimental import pallas as pl
from jax.experimental.pallas import tpu as pltpu
from jax.experimental.pallas import tpu_sc as plsc

N = 10000
NPAD = 10240
IN_DIM = 128
HID = 64
HHID = HID // 2
OUT = 16
LANE = 128

NSC = 2
NTILE = 16
NW = NSC * NTILE
CHUNK = 128
E = 320000
ECH = E // CHUNK
EPT_CHUNKS = 80
EPAD = EPT_CHUNKS * CHUNK * NW
NCH = EPAD // CHUNK
PCH = NCH - ECH
CPT = NCH // NTILE
ROWS_PT = NPAD // NTILE
GROWS_PT = N // NTILE
NBUF = 8
DEPTH = 4

BLK = 1000
GRID = N // BLK


def _vsc_mesh():
    return plsc.VectorSubcoreMesh(core_axis_name="c", subcore_axis_name="s")


_SC_PARAMS = pltpu.CompilerParams(use_tc_tiling_on_sc=False)


def _sc_degree(em, dstpad):
    @functools.partial(
        pl.kernel,
        out_type=jax.ShapeDtypeStruct((NPAD, LANE), jnp.float32),
        mesh=_vsc_mesh(),
        scratch_types=[
            pltpu.VMEM((EPT_CHUNKS, CHUNK), jnp.int32),
            pltpu.VMEM((CHUNK, 16), jnp.float32),
            pltpu.VMEM_SHARED((NPAD, 16), jnp.float32),
            pltpu.SemaphoreType.DMA((NBUF,)),
        ],
        compiler_params=_SC_PARAMS,
    )
    def k(em_hbm, pad_hbm, out_hbm, idx_d, ones, acc, sem_s):
        c = lax.axis_index("c")
        s = lax.axis_index("s")
        wid = c * NTILE + s

        @pl.when(wid < NW - 1)
        def _():
            pltpu.sync_copy(em_hbm.at[1, pl.ds(wid * EPT_CHUNKS, EPT_CHUNKS)],
                            idx_d)

        @pl.when(wid == NW - 1)
        def _():
            main = ECH - (NW - 1) * EPT_CHUNKS
            pltpu.sync_copy(em_hbm.at[1, pl.ds((NW - 1) * EPT_CHUNKS, main)],
                            idx_d.at[pl.ds(0, main)])
            pltpu.sync_copy(pad_hbm, idx_d.at[pl.ds(main, PCH)])

        @pl.loop(0, CHUNK)
        def _(i):
            ones[i, :] = jnp.zeros((16,), jnp.float32)

        @pl.loop(0, ROWS_PT // CHUNK)
        def _(r):
            pltpu.sync_copy(ones.at[:], acc.at[pl.ds(s * ROWS_PT + r * CHUNK, CHUNK)])

        @pl.loop(0, CHUNK)
        def _(i):
            ones[i, :] = jnp.ones((16,), jnp.float32)

        plsc.subcore_barrier()

        def start_scatter(slot, ci):
            pltpu.async_copy(ones, acc.at[idx_d.at[ci]], sem_s.at[slot],
                             add=True)

        def wait_scatter(slot, ci):
            pltpu.make_async_copy(ones, acc.at[idx_d.at[ci]],
                                  sem_s.at[slot]).wait()

        @pl.loop(0, EPT_CHUNKS, step=NBUF)
        def _(t):
            for b in range(NBUF):
                ci = t + b

                @pl.when(ci >= NBUF)
                def _():
                    wait_scatter(b, ci)
                start_scatter(b, ci)

        for b in range(NBUF):
            wait_scatter(b, EPT_CHUNKS - NBUF + b)

        plsc.subcore_barrier()
        pltpu.sync_copy(acc.at[pl.ds(s * ROWS_PT, ROWS_PT)],
                        out_hbm.at[pl.ds(s * ROWS_PT, ROWS_PT),
                                   pl.ds(16 * c, 16)])

    return k(em, dstpad)


def _sc_aggregate(g, em, srcpad, dstpad):
    @functools.partial(
        pl.kernel,
        out_type=jax.ShapeDtypeStruct((NPAD, LANE), jnp.float32),
        mesh=_vsc_mesh(),
        scratch_types=[
            pltpu.VMEM((CPT, CHUNK), jnp.int32),
            pltpu.VMEM((CPT, CHUNK), jnp.int32),
            pltpu.VMEM((NBUF, CHUNK, HHID), jnp.float32),
            pltpu.VMEM((16, HHID), jnp.float32),
            pltpu.VMEM((ROWS_PT, 16), jnp.float32),
            pltpu.VMEM_SHARED((NPAD, HHID), jnp.float32),
            pltpu.VMEM_SHARED((N, HHID), jnp.float32),
            pltpu.SemaphoreType.DMA((NBUF,)),
            pltpu.SemaphoreType.DMA((NBUF,)),
        ],
        compiler_params=_SC_PARAMS,
    )
    def k(g_hbm, em_hbm, spad_hbm, dpad_hbm, out_hbm, idx_s, idx_d, rows,
          zbuf, dbuf, acc, gtab, sem_g, sem_s):
        c = lax.axis_index("c")
        s = lax.axis_index("s")
        col = c * HHID

        @pl.when(s < NTILE - 1)
        def _():
            pltpu.sync_copy(em_hbm.at[0, pl.ds(s * CPT, CPT)], idx_s)
            pltpu.sync_copy(em_hbm.at[1, pl.ds(s * CPT, CPT)], idx_d)

        @pl.when(s == NTILE - 1)
        def _():
            main = ECH - (NTILE - 1) * CPT
            base = (NTILE - 1) * CPT
            pltpu.sync_copy(em_hbm.at[0, pl.ds(base, main)],
                            idx_s.at[pl.ds(0, main)])
            pltpu.sync_copy(em_hbm.at[1, pl.ds(base, main)],
                            idx_d.at[pl.ds(0, main)])
            pltpu.sync_copy(spad_hbm, idx_s.at[pl.ds(main, PCH)])
            pltpu.sync_copy(dpad_hbm, idx_d.at[pl.ds(main, PCH)])

        pltpu.sync_copy(g_hbm.at[pl.ds(s * GROWS_PT, GROWS_PT),
                                 pl.ds(col, HHID)],
                        gtab.at[pl.ds(s * GROWS_PT, GROWS_PT)])

        @pl.loop(0, 16)
        def _(i):
            @pl.loop(0, HHID // 16)
            def _(j):
                zbuf[i, pl.ds(j * 16, 16)] = jnp.zeros((16,), jnp.float32)

        @pl.when(s < NTILE - 1)
        def _():
            pltpu.sync_copy(g_hbm.at[pl.ds(s * ROWS_PT, ROWS_PT),
                                     pl.ds(col, HHID)],
                            acc.at[pl.ds(s * ROWS_PT, ROWS_PT)])

        @pl.when(s == NTILE - 1)
        def _():
            last = (NTILE - 1) * ROWS_PT
            pltpu.sync_copy(g_hbm.at[pl.ds(last, N - last), pl.ds(col, HHID)],
                            acc.at[pl.ds(last, N - last)])

            @pl.loop(N, NPAD, step=16)
            def _(r):
                pltpu.sync_copy(zbuf, acc.at[pl.ds(r, 16)])

        @pl.when(jnp.logical_and(c == 0, s < NTILE - 1))
        def _():
            pltpu.sync_copy(g_hbm.at[pl.ds(s * ROWS_PT, ROWS_PT),
                                     pl.ds(HID, 16)], dbuf)
            pltpu.sync_copy(dbuf, out_hbm.at[pl.ds(s * ROWS_PT, ROWS_PT),
                                             pl.ds(HID, 16)])

        @pl.when(jnp.logical_and(c == 0, s == NTILE - 1))
        def _():
            last = (NTILE - 1) * ROWS_PT
            pltpu.sync_copy(g_hbm.at[pl.ds(last, N - last), pl.ds(HID, 16)],
                            dbuf.at[pl.ds(0, N - last)])
            pltpu.sync_copy(dbuf.at[pl.ds(0, N - last)],
                            out_hbm.at[pl.ds(last, N - last), pl.ds(HID, 16)])

        plsc.subcore_barrier()

        def start_gather(slot, ci):
            pltpu.async_copy(gtab.at[idx_s.at[ci]], rows.at[slot],
                             sem_g.at[slot])

        def wait_gather(slot, ci):
            pltpu.make_async_copy(gtab.at[idx_s.at[ci]], rows.at[slot],
                                  sem_g.at[slot]).wait()

        def start_scatter(slot, ci):
            pltpu.async_copy(rows.at[slot], acc.at[idx_d.at[ci]],
                             sem_s.at[slot], add=True)

        def wait_scatter(slot, ci):
            pltpu.make_async_copy(rows.at[slot], acc.at[idx_d.at[ci]],
                                  sem_s.at[slot]).wait()

        for b in range(DEPTH):
            start_gather(b, b)

        @pl.loop(0, CPT, step=NBUF)
        def _(t):
            for b in range(NBUF):
                ci = t + b
                wait_gather(b, ci)
                start_scatter(b, ci)
                cg = ci + DEPTH
                slot_g = (b + DEPTH) % NBUF

                @pl.when(cg < CPT)
                def _():
                    @pl.when(cg >= NBUF)
                    def _():
                        wait_scatter(slot_g, ci)
                    start_gather(slot_g, cg)

        for b in range(NBUF):
            wait_scatter(b, CPT - NBUF + b)

        plsc.subcore_barrier()
        pltpu.sync_copy(acc.at[pl.ds(s * ROWS_PT, ROWS_PT)],
                        out_hbm.at[pl.ds(s * ROWS_PT, ROWS_PT),
                                   pl.ds(col, HHID)])

    return k(g, em, srcpad, dstpad)


_LANE_SPEC = pl.BlockSpec((BLK, LANE), lambda i: (i, 0))


def _tc_matmul_scale(x, w, degp):
    K, M = w.shape

    def body(x_ref, w_ref, d_ref, o_ref):
        deg = d_ref[:, 0:1] + d_ref[:, 16:17] + 1.0
        dis = lax.rsqrt(deg)
        o_ref[:, :M] = jnp.dot(x_ref[...], w_ref[...],
                               preferred_element_type=jnp.float32) * dis
        o_ref[:, M:M + 1] = dis

    return pl.pallas_call(
        body,
        grid=(GRID,),
        in_specs=[pl.BlockSpec((BLK, K), lambda i: (i, 0)),
                  pl.BlockSpec((K, M), lambda i: (0, 0)),
                  _LANE_SPEC],
        out_specs=_LANE_SPEC,
        out_shape=jax.ShapeDtypeStruct((N, LANE), jnp.float32),
    )(x, w, degp)


def _tc_combine_matmul_scale(u, b, w):
    def body(u_ref, b_ref, w_ref, o_ref):
        dis = u_ref[:, HID:HID + 1]
        t = jnp.maximum(dis * u_ref[:, :HID] + b_ref[...], 0.0)
        o_ref[:, :HID] = jnp.dot(t, w_ref[...],
                                 preferred_element_type=jnp.float32) * dis
        o_ref[:, HID:HID + 1] = dis

    return pl.pallas_call(
        body,
        grid=(GRID,),
        in_specs=[_LANE_SPEC,
                  pl.BlockSpec((1, HID), lambda i: (0, 0)),
                  pl.BlockSpec((HID, HID), lambda i: (0, 0))],
        out_specs=_LANE_SPEC,
        out_shape=jax.ShapeDtypeStruct((N, LANE), jnp.float32),
    )(u, b, w)


def _tc_final(u, b, wf, bf):
    def body(u_ref, b_ref, w_ref, bf_ref, o_ref):
        dis = u_ref[:, HID:HID + 1]
        t = jnp.maximum(dis * u_ref[:, :HID] + b_ref[...], 0.0)
        logits = jnp.dot(t, w_ref[...],
                         preferred_element_type=jnp.float32) + bf_ref[...]
        m = jnp.max(logits, axis=1, keepdims=True)
        lse = jnp.log(jnp.sum(jnp.exp(logits - m), axis=1, keepdims=True)) + m
        o_ref[...] = logits - lse

    return pl.pallas_call(
        body,
        grid=(GRID,),
        in_specs=[_LANE_SPEC,
                  pl.BlockSpec((1, HID), lambda i: (0, 0)),
                  pl.BlockSpec((HID, OUT), lambda i: (0, 0)),
                  pl.BlockSpec((1, OUT), lambda i: (0, 0))],
        out_specs=pl.BlockSpec((BLK, OUT), lambda i: (i, 0)),
        out_shape=jax.ShapeDtypeStruct((N, OUT), jnp.float32),
    )(u, b, wf, bf)


def kernel(x, edge_index, W1, b1, W2, b2, Wf, bf):
    em = edge_index.reshape(2, ECH, CHUNK)
    srcpad = jnp.zeros((PCH, CHUNK), jnp.int32)
    dstpad = (N + (jnp.arange(PCH * CHUNK, dtype=jnp.int32) % (NPAD - N))
              ).reshape(PCH, CHUNK)

    degp = _sc_degree(em, dstpad)
    g1 = _tc_matmul_scale(x, W1, degp)

    u1 = _sc_aggregate(g1, em, srcpad, dstpad)
    g2 = _tc_combine_matmul_scale(u1, b1.reshape(1, HID), W2)

    u2 = _sc_aggregate(g2, em, srcpad, dstpad)
    return _tc_final(u2, b2.reshape(1, HID), Wf, bf.reshape(1, OUT))
